# Initial kernel scaffold; baseline (speedup 1.0000x reference)
import jax
import jax.numpy as jnp
from jax.experimental import pallas as pl


def kernel(l_full_embs, ssf_x, padding_mask, W1, b1, ln_g, ln_b, conv_w, conv_b, ssf_weight, ssf_bias, gate_logit):
    raise NotImplementedError("write your pallas kernel here")


# speedup vs baseline: 9.6038x; 9.6038x over previous
import functools

import jax
import jax.numpy as jnp
from jax import lax
from jax.experimental import pallas as pl
from jax.experimental.pallas import tpu as pltpu
from jax.experimental.pallas import tpu_sc as plsc

B, T, D = 16, 8192, 512
H = D // 8
K = 1024
WIN = 7

TB = 1024
NT = T // TB

NC, NS = 2, 16
NW = NC * NS
HALF = K // 2
CH = 64
NCH = HALF // CH


def _tc_body(mf_ref, ssf_ref, W1_ref, b1_ref, lng_ref, lnb_ref, cw_ref,
             cb_ref, sw_ref, sb_ref, gate_ref, x_ref,
             attn_ref, vmeta_ref, rmeta_ref, c_scr):
    t = pl.program_id(1)

    x = x_ref[0]
    h = jnp.dot(x, W1_ref[...], preferred_element_type=jnp.float32)
    h = h + b1_ref[...]
    h = h * 0.5 * (1.0 + lax.erf(h * (2.0 ** -0.5)))
    mu = jnp.mean(h, axis=-1, keepdims=True)
    var = jnp.mean((h - mu) ** 2, axis=-1, keepdims=True)
    h = (h - mu) * lax.rsqrt(var + 1e-5) * lng_ref[...] + lnb_ref[...]

    c8 = jnp.dot(h, cw_ref[...], preferred_element_type=jnp.float32)
    sx = ssf_ref[0].astype(jnp.bfloat16).astype(jnp.float32)
    sw = sw_ref[...].astype(jnp.bfloat16).astype(jnp.float32)
    wssf = jnp.sum(sx * sw, axis=-1, keepdims=True) + sb_ref[0, 0]
    col7 = (lax.broadcasted_iota(jnp.int32, (1, 8), 1) == 7).astype(jnp.float32)
    c_scr[pl.ds(t * TB, TB), :] = c8 + wssf * col7

    @pl.when(t == NT - 1)
    def _finalize():
        cs = c_scr[...]
        zero3 = jnp.zeros((3, 8), jnp.float32)
        cp = jnp.concatenate([zero3, cs, zero3], axis=0)
        wconv = cb_ref[0, 0]
        for dt in range(WIN):
            wconv = wconv + cp[dt:dt + T, dt:dt + 1]
        alpha = jax.nn.sigmoid(gate_ref[0, 0])
        a = jnp.tanh(alpha * wconv + (1.0 - alpha) * cs[:, 7:8])
        m = mf_ref[0] > 0
        a = jnp.where(m, a, -jnp.inf)
        amax = jnp.max(a)
        e = jnp.exp(a - amax)
        attn = e / jnp.sum(e)
        attn_ref[0] = attn

        enc = jnp.where(m, lax.bitcast_convert_type(attn, jnp.int32),
                        jnp.int32(-1))

        def bis(_, lohi):
            lo, hi = lohi
            mid = lo + (hi - lo) // 2 + 1
            cnt = jnp.sum((enc >= mid).astype(jnp.int32))
            take = cnt >= K
            return (jnp.where(take, mid, lo), jnp.where(take, hi, mid - 1))

        v, _ = lax.fori_loop(0, 32, bis, (jnp.int32(0), jnp.int32(2**31 - 1)))
        r = K - jnp.sum((enc > v).astype(jnp.int32))
        vf = lax.bitcast_convert_type(v, jnp.float32)
        vmeta_ref[...] = jnp.zeros((1, 1, 16), jnp.float32) + vf
        rmeta_ref[...] = jnp.zeros((1, 1, 16), jnp.int32) + r


def _tc_scores(mf, ssf_x, W1, b1, ln_g, ln_b, cw8, conv_b, sw, ssf_bias,
               gate_logit, x):
    grid = (B, NT)
    return pl.pallas_call(
        _tc_body,
        grid=grid,
        in_specs=[
            pl.BlockSpec((1, T, 1), lambda b, t: (b, 0, 0)),
            pl.BlockSpec((1, TB, WIN), lambda b, t: (b, t, 0)),
            pl.BlockSpec((D, H), lambda b, t: (0, 0)),
            pl.BlockSpec((1, H), lambda b, t: (0, 0)),
            pl.BlockSpec((1, H), lambda b, t: (0, 0)),
            pl.BlockSpec((1, H), lambda b, t: (0, 0)),
            pl.BlockSpec((H, 8), lambda b, t: (0, 0)),
            pl.BlockSpec((1, 1), lambda b, t: (0, 0)),
            pl.BlockSpec((1, WIN), lambda b, t: (0, 0)),
            pl.BlockSpec((1, 1), lambda b, t: (0, 0)),
            pl.BlockSpec((1, 1), lambda b, t: (0, 0)),
            pl.BlockSpec((1, TB, D), lambda b, t: (b, t, 0)),
        ],
        out_specs=[
            pl.BlockSpec((1, T, 1), lambda b, t: (b, 0, 0)),
            pl.BlockSpec((1, 1, 16), lambda b, t: (b, 0, 0)),
            pl.BlockSpec((1, 1, 16), lambda b, t: (b, 0, 0)),
        ],
        out_shape=[
            jax.ShapeDtypeStruct((B, T, 1), jnp.float32),
            jax.ShapeDtypeStruct((B, 1, 16), jnp.float32),
            jax.ShapeDtypeStruct((B, 1, 16), jnp.int32),
        ],
        scratch_shapes=[pltpu.VMEM((T, 8), jnp.float32)],
    )(mf, ssf_x, W1, b1, ln_g, ln_b, cw8, conv_b, sw, ssf_bias, gate_logit, x)


def _sc_body(x_hbm, attn_hbm, vmeta_hbm, rmeta_hbm, out_hbm,
             att_v, v_v, r_v, idx_v, buf0, buf1, sem0, sem1):
    cid = lax.axis_index("c")
    sid = lax.axis_index("s")
    wid = sid * NC + cid
    b = wid // 2
    half = wid % 2

    pltpu.sync_copy(attn_hbm.at[b], att_v)
    pltpu.sync_copy(vmeta_hbm.at[b], v_v)
    pltpu.sync_copy(rmeta_hbm.at[b], r_v)
    v = v_v[...]
    r = r_v[...]

    def chunk(i, carry):
        off, eqc = carry
        av = att_v[pl.ds(i * 16, 16)]
        gt = av > v
        eq = av == v
        eq_i = eq.astype(jnp.int32)
        rank = plsc.cumsum(eq_i) - 1 + eqc
        sel = jnp.logical_or(gt, jnp.logical_and(eq, rank < r))
        idxv = lax.iota(jnp.int32, 16) + i * 16
        plsc.store_compressed(idx_v.at[pl.ds(off, 16)], idxv, mask=sel)
        off = off + jnp.sum(sel.astype(jnp.int32), axis=0)
        eqc = eqc + jnp.sum(eq_i, axis=0)
        return off, eqc

    lax.fori_loop(0, T // 16, chunk, (jnp.int32(0), jnp.int32(0)))

    base = half * HALF

    def gch(j, _):
        start = base + j * CH
        pltpu.async_copy(x_hbm.at[b].at[idx_v.at[pl.ds(start, CH)]],
                         buf0, sem0).wait()
        pltpu.sync_copy(buf0, out_hbm.at[b, pl.ds(start, CH)])
        return 0

    lax.fori_loop(0, NCH, gch, 0)


@functools.cache
def _make_sc_gather():
    return pl.kernel(
        _sc_body,
        out_type=jax.ShapeDtypeStruct((B, K, D), jnp.float32),
        mesh=plsc.VectorSubcoreMesh(core_axis_name="c", subcore_axis_name="s",
                                    num_cores=NC, num_subcores=NS),
        scratch_types=[
            pltpu.VMEM((T,), jnp.float32),
            pltpu.VMEM((16,), jnp.float32),
            pltpu.VMEM((16,), jnp.int32),
            pltpu.VMEM((K + 16,), jnp.int32),
            pltpu.VMEM((CH, D), jnp.float32),
            pltpu.VMEM((CH, D), jnp.float32),
            pltpu.SemaphoreType.DMA,
            pltpu.SemaphoreType.DMA,
        ],
        compiler_params=pltpu.CompilerParams(needs_layout_passes=False),
    )


def kernel(l_full_embs, ssf_x, padding_mask, W1, b1, ln_g, ln_b, conv_w,
           conv_b, ssf_weight, ssf_bias, gate_logit):
    mf = padding_mask.astype(jnp.float32)[..., None]
    cw8 = jnp.concatenate(
        [conv_w[0, 0].T, jnp.zeros((H, 1), jnp.float32)], axis=1)
    attn, vmeta, rmeta = _tc_scores(
        mf, ssf_x, W1, b1.reshape(1, H), ln_g.reshape(1, H),
        ln_b.reshape(1, H), cw8, conv_b.reshape(1, 1),
        ssf_weight.reshape(1, WIN), ssf_bias.reshape(1, 1),
        gate_logit.reshape(1, 1), l_full_embs)
    pooled = _make_sc_gather()(l_full_embs, attn[:, :, 0],
                               vmeta[:, 0, :], rmeta[:, 0, :])
    return (pooled, attn)


# speedup vs baseline: 26.4039x; 2.7493x over previous
import functools

import jax
import jax.numpy as jnp
from jax import lax
from jax.experimental import pallas as pl
from jax.experimental.pallas import tpu as pltpu
from jax.experimental.pallas import tpu_sc as plsc

B, T, D = 16, 8192, 512
H = D // 8
K = 1024
WIN = 7

TB = 1024
NT = T // TB

NC, NS = 2, 16
NW = NC * NS
HALF = K // 2
CH = 64
NCH = HALF // CH


def _tc_body(mf_ref, ssf_ref, W1_ref, b1_ref, lng_ref, lnb_ref, cw_ref,
             cb_ref, sw_ref, sb_ref, gate_ref, x_ref,
             attn_ref, vmeta_ref, rmeta_ref, c_scr):
    t = pl.program_id(1)

    x = x_ref[0]
    h = jnp.dot(x, W1_ref[...], preferred_element_type=jnp.float32)
    h = h + b1_ref[...]
    h = h * 0.5 * (1.0 + lax.erf(h * (2.0 ** -0.5)))
    mu = jnp.mean(h, axis=-1, keepdims=True)
    var = jnp.mean((h - mu) ** 2, axis=-1, keepdims=True)
    h = (h - mu) * lax.rsqrt(var + 1e-5) * lng_ref[...] + lnb_ref[...]

    c8 = jnp.dot(h, cw_ref[...], preferred_element_type=jnp.float32)
    sx = ssf_ref[0].astype(jnp.bfloat16).astype(jnp.float32)
    sw = sw_ref[...].astype(jnp.bfloat16).astype(jnp.float32)
    wssf = jnp.sum(sx * sw, axis=-1, keepdims=True) + sb_ref[0, 0]
    col7 = (lax.broadcasted_iota(jnp.int32, (1, 8), 1) == 7).astype(jnp.float32)
    c_scr[:, pl.ds(t * TB, TB)] = jnp.transpose(c8 + wssf * col7, (1, 0))

    @pl.when(t == NT - 1)
    def _finalize():
        cs = c_scr[...]
        lane = lax.broadcasted_iota(jnp.int32, (1, T), 1)
        wconv = jnp.zeros((1, T), jnp.float32) + cb_ref[0, 0]
        for dt in range(WIN):
            k = WIN // 2 - dt
            sh = pltpu.roll(cs[dt:dt + 1, :], k % T, axis=1)
            if k > 0:
                sh = jnp.where(lane < k, 0.0, sh)
            elif k < 0:
                sh = jnp.where(lane >= T + k, 0.0, sh)
            wconv = wconv + sh
        alpha = jax.nn.sigmoid(gate_ref[0, 0])
        a = jnp.tanh(alpha * wconv + (1.0 - alpha) * cs[7:8, :])
        m = mf_ref[0] > 0
        a = jnp.where(m, a, -jnp.inf)
        amax = jnp.max(a)
        e = jnp.exp(a - amax)
        attn = e / jnp.sum(e)
        attn_ref[0] = attn

        enc = jnp.where(m, lax.bitcast_convert_type(attn, jnp.int32),
                        jnp.int32(-1))

        def bis(_, lohi):
            lo, hi = lohi
            mid = lo + (hi - lo) // 2 + 1
            cnt = jnp.sum((enc >= mid).astype(jnp.int32))
            take = cnt >= K
            return (jnp.where(take, mid, lo), jnp.where(take, hi, mid - 1))

        v, _ = lax.fori_loop(0, 32, bis, (jnp.int32(0), jnp.int32(2**31 - 1)))
        r = K - jnp.sum((enc > v).astype(jnp.int32))
        vf = lax.bitcast_convert_type(v, jnp.float32)
        vmeta_ref[...] = jnp.zeros((1, 1, 16), jnp.float32) + vf
        rmeta_ref[...] = jnp.zeros((1, 1, 16), jnp.int32) + r


def _tc_scores(mf, ssf_x, W1, b1, ln_g, ln_b, cw8, conv_b, sw, ssf_bias,
               gate_logit, x):
    grid = (B, NT)
    return pl.pallas_call(
        _tc_body,
        grid=grid,
        in_specs=[
            pl.BlockSpec((1, 1, T), lambda b, t: (b, 0, 0)),
            pl.BlockSpec((1, TB, WIN), lambda b, t: (b, t, 0)),
            pl.BlockSpec((D, H), lambda b, t: (0, 0)),
            pl.BlockSpec((1, H), lambda b, t: (0, 0)),
            pl.BlockSpec((1, H), lambda b, t: (0, 0)),
            pl.BlockSpec((1, H), lambda b, t: (0, 0)),
            pl.BlockSpec((H, 8), lambda b, t: (0, 0)),
            pl.BlockSpec((1, 1), lambda b, t: (0, 0)),
            pl.BlockSpec((1, WIN), lambda b, t: (0, 0)),
            pl.BlockSpec((1, 1), lambda b, t: (0, 0)),
            pl.BlockSpec((1, 1), lambda b, t: (0, 0)),
            pl.BlockSpec((1, TB, D), lambda b, t: (b, t, 0)),
        ],
        out_specs=[
            pl.BlockSpec((1, 1, T), lambda b, t: (b, 0, 0)),
            pl.BlockSpec((1, 1, 16), lambda b, t: (b, 0, 0)),
            pl.BlockSpec((1, 1, 16), lambda b, t: (b, 0, 0)),
        ],
        out_shape=[
            jax.ShapeDtypeStruct((B, 1, T), jnp.float32),
            jax.ShapeDtypeStruct((B, 1, 16), jnp.float32),
            jax.ShapeDtypeStruct((B, 1, 16), jnp.int32),
        ],
        scratch_shapes=[pltpu.VMEM((8, T), jnp.float32)],
    )(mf, ssf_x, W1, b1, ln_g, ln_b, cw8, conv_b, sw, ssf_bias, gate_logit, x)


def _sc_body(x_hbm, attn_hbm, vmeta_hbm, rmeta_hbm, out_hbm,
             att_v, v_v, r_v, idx_v, buf0, buf1, sem0, sem1):
    cid = lax.axis_index("c")
    sid = lax.axis_index("s")
    wid = sid * NC + cid
    b = wid // 2
    half = wid % 2

    pltpu.sync_copy(attn_hbm.at[b], att_v)
    pltpu.sync_copy(vmeta_hbm.at[b], v_v)
    pltpu.sync_copy(rmeta_hbm.at[b], r_v)
    v = v_v[...]
    r = r_v[...]

    def chunk(i, carry):
        off, eqc = carry
        av = att_v[pl.ds(i * 16, 16)]
        gt = av > v
        eq = av == v
        eq_i = eq.astype(jnp.int32)
        rank = plsc.cumsum(eq_i) - 1 + eqc
        sel = jnp.logical_or(gt, jnp.logical_and(eq, rank < r))
        idxv = lax.iota(jnp.int32, 16) + i * 16
        plsc.store_compressed(idx_v.at[pl.ds(off, 16)], idxv, mask=sel)
        off = off + jnp.sum(sel.astype(jnp.int32), axis=0)
        eqc = eqc + jnp.sum(eq_i, axis=0)
        return off, eqc

    lax.fori_loop(0, T // 16, chunk, (jnp.int32(0), jnp.int32(0)))

    base = half * HALF

    def gch(j, _):
        start = base + j * CH
        pltpu.async_copy(x_hbm.at[b].at[idx_v.at[pl.ds(start, CH)]],
                         buf0, sem0).wait()
        pltpu.sync_copy(buf0, out_hbm.at[b, pl.ds(start, CH)])
        return 0

    lax.fori_loop(0, NCH, gch, 0)


@functools.cache
def _make_sc_gather():
    return pl.kernel(
        _sc_body,
        out_type=jax.ShapeDtypeStruct((B, K, D), jnp.float32),
        mesh=plsc.VectorSubcoreMesh(core_axis_name="c", subcore_axis_name="s",
                                    num_cores=NC, num_subcores=NS),
        scratch_types=[
            pltpu.VMEM((T,), jnp.float32),
            pltpu.VMEM((16,), jnp.float32),
            pltpu.VMEM((16,), jnp.int32),
            pltpu.VMEM((K + 16,), jnp.int32),
            pltpu.VMEM((CH, D), jnp.float32),
            pltpu.VMEM((CH, D), jnp.float32),
            pltpu.SemaphoreType.DMA,
            pltpu.SemaphoreType.DMA,
        ],
        compiler_params=pltpu.CompilerParams(needs_layout_passes=False),
    )


def kernel(l_full_embs, ssf_x, padding_mask, W1, b1, ln_g, ln_b, conv_w,
           conv_b, ssf_weight, ssf_bias, gate_logit):
    mf = padding_mask.astype(jnp.float32)[:, None, :]
    cw8 = jnp.concatenate(
        [conv_w[0, 0].T, jnp.zeros((H, 1), jnp.float32)], axis=1)
    attn, vmeta, rmeta = _tc_scores(
        mf, ssf_x, W1, b1.reshape(1, H), ln_g.reshape(1, H),
        ln_b.reshape(1, H), cw8, conv_b.reshape(1, 1),
        ssf_weight.reshape(1, WIN), ssf_bias.reshape(1, 1),
        gate_logit.reshape(1, 1), l_full_embs)
    pooled = _make_sc_gather()(l_full_embs, attn[:, 0, :],
                               vmeta[:, 0, :], rmeta[:, 0, :])
    return (pooled, attn.reshape(B, T, 1))


# speedup vs baseline: 29.3326x; 1.1109x over previous
import functools

import jax
import jax.numpy as jnp
from jax import lax
from jax.experimental import pallas as pl
from jax.experimental.pallas import tpu as pltpu
from jax.experimental.pallas import tpu_sc as plsc

B, T, D = 16, 8192, 512
H = D // 8
K = 1024
WIN = 7

TB = 2048
NT = T // TB

NC, NS = 2, 16
NW = NC * NS
HALF = K // 2
CH = 64
NCH = HALF // CH


def _tc_body(mf_ref, ssf_ref, W1_ref, b1_ref, lng_ref, lnb_ref, cw_ref,
             cb_ref, sw_ref, sb_ref, gate_ref, x_ref,
             attn_ref, vmeta_ref, rmeta_ref, c_scr):
    t = pl.program_id(1)

    x = x_ref[0]
    h = jnp.dot(x, W1_ref[...], preferred_element_type=jnp.float32)
    h = h + b1_ref[...]
    h = h * 0.5 * (1.0 + lax.erf(h * (2.0 ** -0.5)))
    mu = jnp.mean(h, axis=-1, keepdims=True)
    var = jnp.mean((h - mu) ** 2, axis=-1, keepdims=True)
    h = (h - mu) * lax.rsqrt(var + 1e-5) * lng_ref[...] + lnb_ref[...]

    c8 = jnp.dot(h, cw_ref[...], preferred_element_type=jnp.float32)
    sx = ssf_ref[0].astype(jnp.bfloat16).astype(jnp.float32)
    sw = sw_ref[...].astype(jnp.bfloat16).astype(jnp.float32)
    wssf = jnp.sum(sx * sw, axis=-1, keepdims=True) + sb_ref[0, 0]
    col7 = (lax.broadcasted_iota(jnp.int32, (1, 8), 1) == 7).astype(jnp.float32)
    c_scr[:, pl.ds(t * TB, TB)] = jnp.transpose(c8 + wssf * col7, (1, 0))

    @pl.when(t == NT - 1)
    def _finalize():
        cs = c_scr[...]
        lane = lax.broadcasted_iota(jnp.int32, (1, T), 1)
        wconv = jnp.zeros((1, T), jnp.float32) + cb_ref[0, 0]
        for dt in range(WIN):
            k = WIN // 2 - dt
            sh = pltpu.roll(cs[dt:dt + 1, :], k % T, axis=1)
            if k > 0:
                sh = jnp.where(lane < k, 0.0, sh)
            elif k < 0:
                sh = jnp.where(lane >= T + k, 0.0, sh)
            wconv = wconv + sh
        alpha = jax.nn.sigmoid(gate_ref[0, 0])
        a = jnp.tanh(alpha * wconv + (1.0 - alpha) * cs[7:8, :])
        m = mf_ref[0] > 0
        a = jnp.where(m, a, -jnp.inf)
        amax = jnp.max(a)
        e = jnp.exp(a - amax)
        attn = e / jnp.sum(e)
        attn_ref[0] = attn

        enc = jnp.where(m, lax.bitcast_convert_type(attn, jnp.int32),
                        jnp.int32(-1))

        def bis(_, lohi):
            lo, hi = lohi
            mid = lo + (hi - lo) // 2 + 1
            cnt = jnp.sum((enc >= mid).astype(jnp.int32))
            take = cnt >= K
            return (jnp.where(take, mid, lo), jnp.where(take, hi, mid - 1))

        v, _ = lax.fori_loop(0, 32, bis, (jnp.int32(0), jnp.int32(2**31 - 1)))
        r = K - jnp.sum((enc > v).astype(jnp.int32))
        vf = lax.bitcast_convert_type(v, jnp.float32)
        vmeta_ref[...] = jnp.zeros((1, 1, 16), jnp.float32) + vf
        rmeta_ref[...] = jnp.zeros((1, 1, 16), jnp.int32) + r


def _tc_scores(mf, ssf_x, W1, b1, ln_g, ln_b, cw8, conv_b, sw, ssf_bias,
               gate_logit, x):
    grid = (B, NT)
    return pl.pallas_call(
        _tc_body,
        grid=grid,
        in_specs=[
            pl.BlockSpec((1, 1, T), lambda b, t: (b, 0, 0)),
            pl.BlockSpec((1, TB, WIN), lambda b, t: (b, t, 0)),
            pl.BlockSpec((D, H), lambda b, t: (0, 0)),
            pl.BlockSpec((1, H), lambda b, t: (0, 0)),
            pl.BlockSpec((1, H), lambda b, t: (0, 0)),
            pl.BlockSpec((1, H), lambda b, t: (0, 0)),
            pl.BlockSpec((H, 8), lambda b, t: (0, 0)),
            pl.BlockSpec((1, 1), lambda b, t: (0, 0)),
            pl.BlockSpec((1, WIN), lambda b, t: (0, 0)),
            pl.BlockSpec((1, 1), lambda b, t: (0, 0)),
            pl.BlockSpec((1, 1), lambda b, t: (0, 0)),
            pl.BlockSpec((1, TB, D), lambda b, t: (b, t, 0)),
        ],
        out_specs=[
            pl.BlockSpec((1, 1, T), lambda b, t: (b, 0, 0)),
            pl.BlockSpec((1, 1, 16), lambda b, t: (b, 0, 0)),
            pl.BlockSpec((1, 1, 16), lambda b, t: (b, 0, 0)),
        ],
        out_shape=[
            jax.ShapeDtypeStruct((B, 1, T), jnp.float32),
            jax.ShapeDtypeStruct((B, 1, 16), jnp.float32),
            jax.ShapeDtypeStruct((B, 1, 16), jnp.int32),
        ],
        scratch_shapes=[pltpu.VMEM((8, T), jnp.float32)],
        compiler_params=pltpu.CompilerParams(
            dimension_semantics=("parallel", "arbitrary")),
    )(mf, ssf_x, W1, b1, ln_g, ln_b, cw8, conv_b, sw, ssf_bias, gate_logit, x)


def _sc_body(x_hbm, attn_hbm, vmeta_hbm, rmeta_hbm, out_hbm,
             att_v, v_v, r_v, idx_v, buf0, buf1, sem0, sem1):
    cid = lax.axis_index("c")
    sid = lax.axis_index("s")
    wid = sid * NC + cid
    b = wid // 2
    half = wid % 2

    pltpu.sync_copy(attn_hbm.at[b], att_v)
    pltpu.sync_copy(vmeta_hbm.at[b], v_v)
    pltpu.sync_copy(rmeta_hbm.at[b], r_v)
    v = v_v[...]
    r = r_v[...]

    def chunk(i, carry):
        off, eqc = carry
        av = att_v[pl.ds(i * 16, 16)]
        gt = av > v
        eq = av == v
        eq_i = eq.astype(jnp.int32)
        rank = plsc.cumsum(eq_i) - 1 + eqc
        sel = jnp.logical_or(gt, jnp.logical_and(eq, rank < r))
        idxv = lax.iota(jnp.int32, 16) + i * 16
        plsc.store_compressed(idx_v.at[pl.ds(off, 16)], idxv, mask=sel)
        off = off + jnp.sum(sel.astype(jnp.int32), axis=0)
        eqc = eqc + jnp.sum(eq_i, axis=0)
        return off, eqc

    lax.fori_loop(0, T // 16, chunk, (jnp.int32(0), jnp.int32(0)))

    base = half * HALF

    def gch(j, _):
        start = base + j * CH
        pltpu.async_copy(x_hbm.at[b].at[idx_v.at[pl.ds(start, CH)]],
                         buf0, sem0).wait()
        pltpu.sync_copy(buf0, out_hbm.at[b, pl.ds(start, CH)])
        return 0

    lax.fori_loop(0, NCH, gch, 0)


@functools.cache
def _make_sc_gather():
    return pl.kernel(
        _sc_body,
        out_type=jax.ShapeDtypeStruct((B, K, D), jnp.float32),
        mesh=plsc.VectorSubcoreMesh(core_axis_name="c", subcore_axis_name="s",
                                    num_cores=NC, num_subcores=NS),
        scratch_types=[
            pltpu.VMEM((T,), jnp.float32),
            pltpu.VMEM((16,), jnp.float32),
            pltpu.VMEM((16,), jnp.int32),
            pltpu.VMEM((K + 16,), jnp.int32),
            pltpu.VMEM((CH, D), jnp.float32),
            pltpu.VMEM((CH, D), jnp.float32),
            pltpu.SemaphoreType.DMA,
            pltpu.SemaphoreType.DMA,
        ],
        compiler_params=pltpu.CompilerParams(needs_layout_passes=False),
    )


def kernel(l_full_embs, ssf_x, padding_mask, W1, b1, ln_g, ln_b, conv_w,
           conv_b, ssf_weight, ssf_bias, gate_logit):
    mf = padding_mask.astype(jnp.float32)[:, None, :]
    cw8 = jnp.concatenate(
        [conv_w[0, 0].T, jnp.zeros((H, 1), jnp.float32)], axis=1)
    attn, vmeta, rmeta = _tc_scores(
        mf, ssf_x, W1, b1.reshape(1, H), ln_g.reshape(1, H),
        ln_b.reshape(1, H), cw8, conv_b.reshape(1, 1),
        ssf_weight.reshape(1, WIN), ssf_bias.reshape(1, 1),
        gate_logit.reshape(1, 1), l_full_embs)
    pooled = _make_sc_gather()(l_full_embs, attn[:, 0, :],
                               vmeta[:, 0, :], rmeta[:, 0, :])
    return (pooled, attn.reshape(B, T, 1))


# speedup vs baseline: 31.1811x; 1.0630x over previous
import functools

import jax
import jax.numpy as jnp
from jax import lax
from jax.experimental import pallas as pl
from jax.experimental.pallas import tpu as pltpu
from jax.experimental.pallas import tpu_sc as plsc

B, T, D = 16, 8192, 512
H = D // 8
K = 1024
WIN = 7

TB = 4096
NT = T // TB

NC, NS = 2, 16
NW = NC * NS
HALF = K // 2
CH = 64
NCH = HALF // CH


def _tc_body(mf_ref, ssf_ref, W1_ref, b1_ref, lng_ref, lnb_ref, cw_ref,
             cb_ref, sw_ref, sb_ref, gate_ref, x_ref,
             attn_ref, vmeta_ref, rmeta_ref, c_scr):
    t = pl.program_id(1)

    x = x_ref[0]
    h = jnp.dot(x, W1_ref[...], preferred_element_type=jnp.float32)
    h = h + b1_ref[...]
    h = h * 0.5 * (1.0 + lax.erf(h * (2.0 ** -0.5)))
    mu = jnp.mean(h, axis=-1, keepdims=True)
    var = jnp.mean((h - mu) ** 2, axis=-1, keepdims=True)
    h = (h - mu) * lax.rsqrt(var + 1e-5) * lng_ref[...] + lnb_ref[...]

    c8 = jnp.dot(h, cw_ref[...], preferred_element_type=jnp.float32)
    sx = ssf_ref[0].astype(jnp.bfloat16).astype(jnp.float32)
    sw = sw_ref[...].astype(jnp.bfloat16).astype(jnp.float32)
    wssf = jnp.sum(sx * sw, axis=-1, keepdims=True) + sb_ref[0, 0]
    col7 = (lax.broadcasted_iota(jnp.int32, (1, 8), 1) == 7).astype(jnp.float32)
    c_scr[:, pl.ds(t * TB, TB)] = jnp.transpose(c8 + wssf * col7, (1, 0))

    @pl.when(t == NT - 1)
    def _finalize():
        cs = c_scr[...]
        lane = lax.broadcasted_iota(jnp.int32, (1, T), 1)
        wconv = jnp.zeros((1, T), jnp.float32) + cb_ref[0, 0]
        for dt in range(WIN):
            k = WIN // 2 - dt
            sh = pltpu.roll(cs[dt:dt + 1, :], k % T, axis=1)
            if k > 0:
                sh = jnp.where(lane < k, 0.0, sh)
            elif k < 0:
                sh = jnp.where(lane >= T + k, 0.0, sh)
            wconv = wconv + sh
        alpha = jax.nn.sigmoid(gate_ref[0, 0])
        a = jnp.tanh(alpha * wconv + (1.0 - alpha) * cs[7:8, :])
        m = mf_ref[0] > 0
        a = jnp.where(m, a, -jnp.inf)
        amax = jnp.max(a)
        e = jnp.exp(a - amax)
        attn = e / jnp.sum(e)
        attn_ref[0] = attn

        enc = jnp.where(m, lax.bitcast_convert_type(attn, jnp.int32),
                        jnp.int32(-1))

        def bis(_, lohi):
            lo, hi = lohi
            mid = lo + (hi - lo) // 2 + 1
            cnt = jnp.sum((enc >= mid).astype(jnp.int32))
            take = cnt >= K
            return (jnp.where(take, mid, lo), jnp.where(take, hi, mid - 1))

        v, _ = lax.fori_loop(0, 32, bis, (jnp.int32(0), jnp.int32(2**31 - 1)))
        r = K - jnp.sum((enc > v).astype(jnp.int32))
        vf = lax.bitcast_convert_type(v, jnp.float32)
        vmeta_ref[...] = jnp.zeros((1, 1, 16), jnp.float32) + vf
        rmeta_ref[...] = jnp.zeros((1, 1, 16), jnp.int32) + r


def _tc_scores(mf, ssf_x, W1, b1, ln_g, ln_b, cw8, conv_b, sw, ssf_bias,
               gate_logit, x):
    grid = (B, NT)
    return pl.pallas_call(
        _tc_body,
        grid=grid,
        in_specs=[
            pl.BlockSpec((1, 1, T), lambda b, t: (b, 0, 0)),
            pl.BlockSpec((1, TB, WIN), lambda b, t: (b, t, 0)),
            pl.BlockSpec((D, H), lambda b, t: (0, 0)),
            pl.BlockSpec((1, H), lambda b, t: (0, 0)),
            pl.BlockSpec((1, H), lambda b, t: (0, 0)),
            pl.BlockSpec((1, H), lambda b, t: (0, 0)),
            pl.BlockSpec((H, 8), lambda b, t: (0, 0)),
            pl.BlockSpec((1, 1), lambda b, t: (0, 0)),
            pl.BlockSpec((1, WIN), lambda b, t: (0, 0)),
            pl.BlockSpec((1, 1), lambda b, t: (0, 0)),
            pl.BlockSpec((1, 1), lambda b, t: (0, 0)),
            pl.BlockSpec((1, TB, D), lambda b, t: (b, t, 0)),
        ],
        out_specs=[
            pl.BlockSpec((1, 1, T), lambda b, t: (b, 0, 0)),
            pl.BlockSpec((1, 1, 16), lambda b, t: (b, 0, 0)),
            pl.BlockSpec((1, 1, 16), lambda b, t: (b, 0, 0)),
        ],
        out_shape=[
            jax.ShapeDtypeStruct((B, 1, T), jnp.float32),
            jax.ShapeDtypeStruct((B, 1, 16), jnp.float32),
            jax.ShapeDtypeStruct((B, 1, 16), jnp.int32),
        ],
        scratch_shapes=[pltpu.VMEM((8, T), jnp.float32)],
        compiler_params=pltpu.CompilerParams(
            dimension_semantics=("parallel", "arbitrary")),
    )(mf, ssf_x, W1, b1, ln_g, ln_b, cw8, conv_b, sw, ssf_bias, gate_logit, x)


def _sc_body(x_hbm, attn_hbm, vmeta_hbm, rmeta_hbm, out_hbm,
             att_v, v_v, r_v, idx_v, buf0, buf1, sem0, sem1):
    cid = lax.axis_index("c")
    sid = lax.axis_index("s")
    wid = sid * NC + cid
    b = wid // 2
    half = wid % 2

    pltpu.sync_copy(attn_hbm.at[b], att_v)
    pltpu.sync_copy(vmeta_hbm.at[b], v_v)
    pltpu.sync_copy(rmeta_hbm.at[b], r_v)
    v = v_v[...]
    r = r_v[...]

    def chunk(i, carry):
        off, eqc = carry
        av = att_v[pl.ds(i * 16, 16)]
        gt = av > v
        eq = av == v
        eq_i = eq.astype(jnp.int32)
        rank = plsc.cumsum(eq_i) - 1 + eqc
        sel = jnp.logical_or(gt, jnp.logical_and(eq, rank < r))
        idxv = lax.iota(jnp.int32, 16) + i * 16
        plsc.store_compressed(idx_v.at[pl.ds(off, 16)], idxv, mask=sel)
        off = off + jnp.sum(sel.astype(jnp.int32), axis=0)
        eqc = eqc + jnp.sum(eq_i, axis=0)
        return off, eqc

    lax.fori_loop(0, T // 16, chunk, (jnp.int32(0), jnp.int32(0)))

    base = half * HALF

    def gch(j, _):
        start = base + j * CH
        pltpu.async_copy(x_hbm.at[b].at[idx_v.at[pl.ds(start, CH)]],
                         buf0, sem0).wait()
        pltpu.sync_copy(buf0, out_hbm.at[b, pl.ds(start, CH)])
        return 0

    lax.fori_loop(0, NCH, gch, 0)


@functools.cache
def _make_sc_gather():
    return pl.kernel(
        _sc_body,
        out_type=jax.ShapeDtypeStruct((B, K, D), jnp.float32),
        mesh=plsc.VectorSubcoreMesh(core_axis_name="c", subcore_axis_name="s",
                                    num_cores=NC, num_subcores=NS),
        scratch_types=[
            pltpu.VMEM((T,), jnp.float32),
            pltpu.VMEM((16,), jnp.float32),
            pltpu.VMEM((16,), jnp.int32),
            pltpu.VMEM((K + 16,), jnp.int32),
            pltpu.VMEM((CH, D), jnp.float32),
            pltpu.VMEM((CH, D), jnp.float32),
            pltpu.SemaphoreType.DMA,
            pltpu.SemaphoreType.DMA,
        ],
        compiler_params=pltpu.CompilerParams(needs_layout_passes=False),
    )


def kernel(l_full_embs, ssf_x, padding_mask, W1, b1, ln_g, ln_b, conv_w,
           conv_b, ssf_weight, ssf_bias, gate_logit):
    mf = padding_mask.astype(jnp.float32)[:, None, :]
    cw8 = jnp.concatenate(
        [conv_w[0, 0].T, jnp.zeros((H, 1), jnp.float32)], axis=1)
    attn, vmeta, rmeta = _tc_scores(
        mf, ssf_x, W1, b1.reshape(1, H), ln_g.reshape(1, H),
        ln_b.reshape(1, H), cw8, conv_b.reshape(1, 1),
        ssf_weight.reshape(1, WIN), ssf_bias.reshape(1, 1),
        gate_logit.reshape(1, 1), l_full_embs)
    pooled = _make_sc_gather()(l_full_embs, attn[:, 0, :],
                               vmeta[:, 0, :], rmeta[:, 0, :])
    return (pooled, attn.reshape(B, T, 1))
